# Initial kernel scaffold; baseline (speedup 1.0000x reference)
import jax
import jax.numpy as jnp
from jax.experimental import pallas as pl


def kernel(pop_history, item, time, release_time, category, store, item_table, time_table, cat_table, store_table, gamma_time, beta_time, W_time, b_time, gamma_side, beta_side, W_side, b_side, attn_w):
    raise NotImplementedError("write your pallas kernel here")


# speedup vs baseline: 3.5378x; 3.5378x over previous
import functools
import math

import jax
import jax.numpy as jnp
from jax import lax
from jax.experimental import pallas as pl
from jax.experimental.pallas import tpu as pltpu
from jax.experimental.pallas import tpu_sc as plsc

_ALPHA = 0.2
_EPS = 1e-5
_B = 16384
_D = 64
_T = 200
_BLK = 512
_NB = _B // _BLK
_CH = 128
_L2A = math.log2(1.0 - _ALPHA)


def _sc_gather(idx_all, item_table, time_table, cat_table, store_table):
  info = plsc.get_sparse_core_info()
  nc, ns = info.num_cores, info.num_subcores
  nw = nc * ns
  bpw = _B // nw
  nch = bpw // _CH
  mesh = plsc.VectorSubcoreMesh(core_axis_name="c", subcore_axis_name="s")
  out_type = tuple(
      jax.ShapeDtypeStruct((_B, _D), jnp.float32) for _ in range(5))

  @functools.partial(
      pl.kernel,
      mesh=mesh,
      out_type=out_type,
      compiler_params=pltpu.CompilerParams(use_tc_tiling_on_sc=False),
      scratch_types=[
          pltpu.VMEM((5, nch, _CH), jnp.int32),
          pltpu.VMEM((bpw, _D), jnp.float32),
          pltpu.VMEM((bpw, _D), jnp.float32),
          pltpu.SemaphoreType.DMA,
          pltpu.SemaphoreType.DMA,
      ],
  )
  def gather_kernel(idx_h, it_h, tt_h, ct_h, st_h,
                    o0, o1, o2, o3, o4,
                    idx_v, rows_a, rows_b, sem_a, sem_b):
    wid = lax.axis_index("s") * nc + lax.axis_index("c")
    base = wid * bpw
    tabs = (it_h, tt_h, tt_h, ct_h, st_h)
    outs = (o0, o1, o2, o3, o4)
    bufs = (rows_a, rows_b)
    sems = (sem_a, sem_b)
    pltpu.sync_copy(idx_h.at[wid], idx_v)

    def fire(t):
      buf = bufs[t % 2]
      sem = sems[t % 2]
      handles = []
      for j in range(nch):
        handles.append(
            pltpu.async_copy(tabs[t].at[idx_v.at[t, j]],
                             buf.at[pl.ds(j * _CH, _CH)], sem))
      return handles

    pending = fire(0)
    for t in range(5):
      for h in pending:
        h.wait()
      cur = bufs[t % 2]
      if t + 1 < 5:
        pending = fire(t + 1)
      pltpu.sync_copy(cur, outs[t].at[pl.ds(base, bpw)])

  return gather_kernel(idx_all, item_table, time_table, cat_table,
                       store_table)


def _pop_body(ph_ref, tf_ref, out_ref):
  ph = ph_ref[...]
  tb = jnp.maximum(tf_ref[...] - 1.0, 0.0)
  k = lax.broadcasted_iota(jnp.int32, (_BLK, _T), 1).astype(jnp.float32)
  w = jnp.exp2((tb - k) * _L2A)
  coef = jnp.where(k > tb, 0.0, jnp.where(k == 0.0, w, _ALPHA * w))
  out_ref[...] = jnp.sum(coef * ph, axis=1, keepdims=True)


def _head_body(pop_ref, ie_ref, te_ref, re_ref, ce_ref, se_ref,
               gt_ref, bt_ref, wt_ref, bt0_ref,
               gs_ref, bs_ref, ws_ref, bs0_ref, aw_ref,
               opop_ref, otime_ref, oside_ref, oout_ref, acc_ref):
  p = pl.program_id(0)
  b = pl.program_id(1)

  @pl.when(jnp.logical_and(p == 0, b == 0))
  def _init():
    acc_ref[...] = jnp.zeros_like(acc_ref)

  @pl.when(p == 0)
  def _stats():
    ie = ie_ref[...]
    te = te_ref[...]
    re = re_ref[...]
    ce = ce_ref[...]
    se = se_ref[...]
    gap = re - te
    cols = (ie, te, re, ce, se)
    sums = [jnp.sum(x, axis=0, keepdims=True) for x in cols]
    sqs = [jnp.sum(x * x, axis=0, keepdims=True) for x in cols]
    gsum = jnp.sum(gap, axis=0, keepdims=True)
    gsq = jnp.sum(gap * gap, axis=0, keepdims=True)
    z = jnp.zeros((4, _D), jnp.float32)
    upd = jnp.concatenate(sums + sqs + [gsum, gsq, z], axis=0)
    acc_ref[...] += upd

  @pl.when(p == 1)
  def _final():
    st = acc_ref[...]
    inv_n = 1.0 / _B
    mu = st[0:5] * inv_n
    ex2 = st[5:10] * inv_n
    mu_g = st[10:11] * inv_n
    ex2_g = st[11:12] * inv_n
    var = ex2 - mu * mu
    var_g = ex2_g - mu_g * mu_g
    mu_t = jnp.concatenate([mu_g, mu[0:1], mu[1:2], mu[2:3]], axis=0)
    var_t = jnp.concatenate([var_g, var[0:1], var[1:2], var[2:3]], axis=0)
    w4 = wt_ref[...]
    v4 = gt_ref[...] * w4 * lax.rsqrt(var_t + _EPS)
    c_t = (jnp.sum(bt_ref[...] * w4) - jnp.sum(mu_t * v4)
           + bt0_ref[...])
    w2 = ws_ref[...]
    v2 = gs_ref[...] * w2 * lax.rsqrt(var[3:5] + _EPS)
    c_s = (jnp.sum(bs_ref[...] * w2) - jnp.sum(mu[3:5] * v2)
           + bs0_ref[...])
    aw = aw_ref[...]
    e = jnp.exp(aw - jnp.max(aw))
    wsm = e / jnp.sum(e)
    wa = wsm[0:1, 0:1]
    wb = wsm[0:1, 1:2]
    wc = wsm[0:1, 2:3]

    ie = ie_ref[...]
    te = te_ref[...]
    re = re_ref[...]
    gap = re - te
    dt = jnp.sum(gap * v4[0:1] + ie * v4[1:2] + te * v4[2:3] + re * v4[3:4],
                 axis=1, keepdims=True)
    t_out = jnp.maximum(dt + c_t, 0.0)
    ds = jnp.sum(ce_ref[...] * v2[0:1] + se_ref[...] * v2[1:2],
                 axis=1, keepdims=True)
    s_out = ds + c_s

    w_pop = pop_ref[...] * wa
    w_time = t_out * wb
    w_side = s_out * wc
    opop_ref[...] = w_pop
    otime_ref[...] = w_time
    oside_ref[...] = w_side
    oout_ref[...] = w_pop + w_time + w_side


def kernel(pop_history, item, time, release_time, category, store,
           item_table, time_table, cat_table, store_table,
           gamma_time, beta_time, W_time, b_time,
           gamma_side, beta_side, W_side, b_side, attn_w):
  info = plsc.get_sparse_core_info()
  nw = info.num_cores * info.num_subcores
  bpw = _B // nw
  nch = bpw // _CH

  ids = jnp.stack([item, time, release_time, category, store]).astype(
      jnp.int32)
  idx_all = ids.reshape(5, nw, nch, _CH).transpose(1, 0, 2, 3)
  item_e, time_e, rel_e, cat_e, store_e = _sc_gather(
      idx_all, item_table, time_table, cat_table, store_table)

  tf = time.astype(jnp.float32).reshape(_B, 1)
  pop_raw = pl.pallas_call(
      _pop_body,
      grid=(_NB,),
      in_specs=[
          pl.BlockSpec((_BLK, _T), lambda b: (b, 0)),
          pl.BlockSpec((_BLK, 1), lambda b: (b, 0)),
      ],
      out_specs=pl.BlockSpec((_BLK, 1), lambda b: (b, 0)),
      out_shape=jax.ShapeDtypeStruct((_B, 1), jnp.float32),
  )(pop_history, tf)

  gt = gamma_time.reshape(4, _D)
  bt = beta_time.reshape(4, _D)
  wt = W_time.reshape(4, _D)
  bt0 = b_time.reshape(1, 1)
  gs = gamma_side.reshape(2, _D)
  bs = beta_side.reshape(2, _D)
  ws = W_side.reshape(2, _D)
  bs0 = b_side.reshape(1, 1)
  aw = attn_w.reshape(1, 3)

  blk = lambda p, b: (b, 0)
  full = lambda p, b: (0, 0)
  espec = pl.BlockSpec((_BLK, _D), blk)
  out_spec = pl.BlockSpec((_BLK, 1), blk)
  w_pop, w_time, w_side, output = pl.pallas_call(
      _head_body,
      grid=(2, _NB),
      in_specs=[
          pl.BlockSpec((_BLK, 1), blk),
          espec, espec, espec, espec, espec,
          pl.BlockSpec((4, _D), full),
          pl.BlockSpec((4, _D), full),
          pl.BlockSpec((4, _D), full),
          pl.BlockSpec((1, 1), full),
          pl.BlockSpec((2, _D), full),
          pl.BlockSpec((2, _D), full),
          pl.BlockSpec((2, _D), full),
          pl.BlockSpec((1, 1), full),
          pl.BlockSpec((1, 3), full),
      ],
      out_specs=[out_spec, out_spec, out_spec, out_spec],
      out_shape=[jax.ShapeDtypeStruct((_B, 1), jnp.float32)] * 4,
      scratch_shapes=[pltpu.VMEM((16, _D), jnp.float32)],
  )(pop_raw, item_e, time_e, rel_e, cat_e, store_e,
    gt, bt, wt, bt0, gs, bs, ws, bs0, aw)

  return (w_pop, w_time, w_side, output)


# speedup vs baseline: 3.6407x; 1.0291x over previous
import functools
import math

import jax
import jax.numpy as jnp
from jax import lax
from jax.experimental import pallas as pl
from jax.experimental.pallas import tpu as pltpu
from jax.experimental.pallas import tpu_sc as plsc

_ALPHA = 0.2
_EPS = 1e-5
_B = 16384
_D = 64
_T = 200
_BLK = 512
_NB = _B // _BLK
_CH = 128
_L2A = math.log2(1.0 - _ALPHA)


def _sc_gather(item, time_i, rel, cat, store,
               item_table, time_table, cat_table, store_table):
  info = plsc.get_sparse_core_info()
  nc, ns = info.num_cores, info.num_subcores
  nw = nc * ns
  bpw = _B // nw
  nch = bpw // _CH
  mesh = plsc.VectorSubcoreMesh(core_axis_name="c", subcore_axis_name="s")
  out_type = tuple(
      jax.ShapeDtypeStruct((_B, _D), jnp.float32) for _ in range(5))

  @functools.partial(
      pl.kernel,
      mesh=mesh,
      out_type=out_type,
      compiler_params=pltpu.CompilerParams(use_tc_tiling_on_sc=False),
      scratch_types=[
          pltpu.VMEM((5, bpw), jnp.int32),
          pltpu.VMEM((bpw, _D), jnp.float32),
          pltpu.VMEM((bpw, _D), jnp.float32),
          pltpu.SemaphoreType.DMA,
          pltpu.SemaphoreType.DMA,
      ],
  )
  def gather_kernel(i0_h, i1_h, i2_h, i3_h, i4_h, it_h, tt_h, ct_h, st_h,
                    o0, o1, o2, o3, o4,
                    idx_v, rows_a, rows_b, sem_a, sem_b):
    wid = lax.axis_index("s") * nc + lax.axis_index("c")
    base = wid * bpw
    idxs = (i0_h, i1_h, i2_h, i3_h, i4_h)
    tabs = (it_h, tt_h, tt_h, ct_h, st_h)
    outs = (o0, o1, o2, o3, o4)
    bufs = (rows_a, rows_b)
    sems = (sem_a, sem_b)
    for t in range(5):
      pltpu.sync_copy(idxs[t].at[pl.ds(base, bpw)], idx_v.at[t])

    def fire(t):
      buf = bufs[t % 2]
      sem = sems[t % 2]
      handles = []
      for j in range(nch):
        handles.append(
            pltpu.async_copy(tabs[t].at[idx_v.at[t, pl.ds(j * _CH, _CH)]],
                             buf.at[pl.ds(j * _CH, _CH)], sem))
      return handles

    pending = fire(0)
    for t in range(5):
      for h in pending:
        h.wait()
      cur = bufs[t % 2]
      if t + 1 < 5:
        pending = fire(t + 1)
      pltpu.sync_copy(cur, outs[t].at[pl.ds(base, bpw)])

  return gather_kernel(item, time_i, rel, cat, store,
                       item_table, time_table, cat_table, store_table)


def _pop_body(ph_ref, tf_ref, out_ref):
  ph = ph_ref[...]
  tb = jnp.maximum(tf_ref[...] - 1.0, 0.0)
  k = lax.broadcasted_iota(jnp.int32, (1, _T), 1).astype(jnp.float32)
  row = jnp.exp2(tb * _L2A)
  col = jnp.exp2(k * (-_L2A))
  w = row * col
  coef = jnp.where(k > tb, 0.0, jnp.where(k == 0.0, w, _ALPHA * w))
  out_ref[...] = jnp.sum(coef * ph, axis=1, keepdims=True)


def _stats_body(ie_ref, te_ref, re_ref, ce_ref, se_ref, out_ref, acc_ref):
  b = pl.program_id(0)

  @pl.when(b == 0)
  def _init():
    acc_ref[...] = jnp.zeros_like(acc_ref)

  ie = ie_ref[...]
  te = te_ref[...]
  re = re_ref[...]
  ce = ce_ref[...]
  se = se_ref[...]
  gap = re - te
  cols = (ie, te, re, ce, se)
  sums = [jnp.sum(x, axis=0, keepdims=True) for x in cols]
  sqs = [jnp.sum(x * x, axis=0, keepdims=True) for x in cols]
  gsum = jnp.sum(gap, axis=0, keepdims=True)
  gsq = jnp.sum(gap * gap, axis=0, keepdims=True)
  z = jnp.zeros((4, _D), jnp.float32)
  acc_ref[...] += jnp.concatenate(sums + sqs + [gsum, gsq, z], axis=0)

  @pl.when(b == _NB - 1)
  def _emit():
    out_ref[...] = acc_ref[...]


def _final_body(st_ref, pop_ref, ie_ref, te_ref, re_ref, ce_ref, se_ref,
                gt_ref, bt_ref, wt_ref, bt0_ref,
                gs_ref, bs_ref, ws_ref, bs0_ref, aw_ref,
                opop_ref, otime_ref, oside_ref, oout_ref):
  st = st_ref[...]
  inv_n = 1.0 / _B
  mu = st[0:5] * inv_n
  ex2 = st[5:10] * inv_n
  mu_g = st[10:11] * inv_n
  ex2_g = st[11:12] * inv_n
  var = ex2 - mu * mu
  var_g = ex2_g - mu_g * mu_g
  mu_t = jnp.concatenate([mu_g, mu[0:1], mu[1:2], mu[2:3]], axis=0)
  var_t = jnp.concatenate([var_g, var[0:1], var[1:2], var[2:3]], axis=0)
  w4 = wt_ref[...]
  v4 = gt_ref[...] * w4 * lax.rsqrt(var_t + _EPS)
  c_t = jnp.sum(bt_ref[...] * w4) - jnp.sum(mu_t * v4) + bt0_ref[...]
  w2 = ws_ref[...]
  v2 = gs_ref[...] * w2 * lax.rsqrt(var[3:5] + _EPS)
  c_s = jnp.sum(bs_ref[...] * w2) - jnp.sum(mu[3:5] * v2) + bs0_ref[...]
  aw = aw_ref[...]
  e = jnp.exp(aw - jnp.max(aw))
  wsm = e / jnp.sum(e)
  wa = wsm[0:1, 0:1]
  wb = wsm[0:1, 1:2]
  wc = wsm[0:1, 2:3]

  ie = ie_ref[...]
  te = te_ref[...]
  re = re_ref[...]
  gap = re - te
  dt = jnp.sum(gap * v4[0:1] + ie * v4[1:2] + te * v4[2:3] + re * v4[3:4],
               axis=1, keepdims=True)
  t_out = jnp.maximum(dt + c_t, 0.0)
  ds = jnp.sum(ce_ref[...] * v2[0:1] + se_ref[...] * v2[1:2],
               axis=1, keepdims=True)
  s_out = ds + c_s

  w_pop = pop_ref[...] * wa
  w_time = t_out * wb
  w_side = s_out * wc
  opop_ref[...] = w_pop
  otime_ref[...] = w_time
  oside_ref[...] = w_side
  oout_ref[...] = w_pop + w_time + w_side


def kernel(pop_history, item, time, release_time, category, store,
           item_table, time_table, cat_table, store_table,
           gamma_time, beta_time, W_time, b_time,
           gamma_side, beta_side, W_side, b_side, attn_w):
  i32 = jnp.int32
  item_e, time_e, rel_e, cat_e, store_e = _sc_gather(
      item.astype(i32), time.astype(i32), release_time.astype(i32),
      category.astype(i32), store.astype(i32),
      item_table, time_table, cat_table, store_table)

  tf = time.astype(jnp.float32).reshape(_B, 1)
  pop_raw = pl.pallas_call(
      _pop_body,
      grid=(_NB,),
      in_specs=[
          pl.BlockSpec((_BLK, _T), lambda b: (b, 0)),
          pl.BlockSpec((_BLK, 1), lambda b: (b, 0)),
      ],
      out_specs=pl.BlockSpec((_BLK, 1), lambda b: (b, 0)),
      out_shape=jax.ShapeDtypeStruct((_B, 1), jnp.float32),
  )(pop_history, tf)

  blk = lambda b: (b, 0)
  full = lambda b: (0, 0)
  espec = pl.BlockSpec((_BLK, _D), blk)
  stats = pl.pallas_call(
      _stats_body,
      grid=(_NB,),
      in_specs=[espec] * 5,
      out_specs=pl.BlockSpec((16, _D), full),
      out_shape=jax.ShapeDtypeStruct((16, _D), jnp.float32),
      scratch_shapes=[pltpu.VMEM((16, _D), jnp.float32)],
  )(item_e, time_e, rel_e, cat_e, store_e)

  gt = gamma_time.reshape(4, _D)
  bt = beta_time.reshape(4, _D)
  wt = W_time.reshape(4, _D)
  bt0 = b_time.reshape(1, 1)
  gs = gamma_side.reshape(2, _D)
  bs = beta_side.reshape(2, _D)
  ws = W_side.reshape(2, _D)
  bs0 = b_side.reshape(1, 1)
  aw = attn_w.reshape(1, 3)

  out_spec = pl.BlockSpec((_BLK, 1), blk)
  w_pop, w_time, w_side, output = pl.pallas_call(
      _final_body,
      grid=(_NB,),
      in_specs=[
          pl.BlockSpec((16, _D), full),
          pl.BlockSpec((_BLK, 1), blk),
          espec, espec, espec, espec, espec,
          pl.BlockSpec((4, _D), full),
          pl.BlockSpec((4, _D), full),
          pl.BlockSpec((4, _D), full),
          pl.BlockSpec((1, 1), full),
          pl.BlockSpec((2, _D), full),
          pl.BlockSpec((2, _D), full),
          pl.BlockSpec((2, _D), full),
          pl.BlockSpec((1, 1), full),
          pl.BlockSpec((1, 3), full),
      ],
      out_specs=[out_spec] * 4,
      out_shape=[jax.ShapeDtypeStruct((_B, 1), jnp.float32)] * 4,
  )(stats, pop_raw, item_e, time_e, rel_e, cat_e, store_e,
    gt, bt, wt, bt0, gs, bs, ws, bs0, aw)

  return (w_pop, w_time, w_side, output)


# speedup vs baseline: 5.8716x; 1.6128x over previous
import functools
import math

import jax
import jax.numpy as jnp
from jax import lax
from jax.experimental import pallas as pl
from jax.experimental.pallas import tpu as pltpu
from jax.experimental.pallas import tpu_sc as plsc

_ALPHA = 0.2
_EPS = 1e-5
_B = 16384
_D = 64
_T = 200
_BLK = 1024
_NB = _B // _BLK
_PR = _BLK // 2
_OR = _BLK // 128
_CH = 128
_L2A = math.log2(1.0 - _ALPHA)


def _sc_gather(item, time_i, rel, cat, store,
               item_table, time_table, cat_table, store_table):
  info = plsc.get_sparse_core_info()
  nc, ns = info.num_cores, info.num_subcores
  nw = nc * ns
  bpw = _B // nw
  nch = bpw // _CH
  mesh = plsc.VectorSubcoreMesh(core_axis_name="c", subcore_axis_name="s")
  out_type = tuple(
      jax.ShapeDtypeStruct((_B, _D), jnp.float32) for _ in range(5))

  @functools.partial(
      pl.kernel,
      mesh=mesh,
      out_type=out_type,
      compiler_params=pltpu.CompilerParams(use_tc_tiling_on_sc=False),
      scratch_types=[
          pltpu.VMEM((5, bpw), jnp.int32),
          pltpu.VMEM((bpw, _D), jnp.float32),
          pltpu.VMEM((bpw, _D), jnp.float32),
          pltpu.SemaphoreType.DMA,
          pltpu.SemaphoreType.DMA,
      ],
  )
  def gather_kernel(i0_h, i1_h, i2_h, i3_h, i4_h, it_h, tt_h, ct_h, st_h,
                    o0, o1, o2, o3, o4,
                    idx_v, rows_a, rows_b, sem_a, sem_b):
    wid = lax.axis_index("s") * nc + lax.axis_index("c")
    base = wid * bpw
    idxs = (i0_h, i1_h, i2_h, i3_h, i4_h)
    tabs = (it_h, tt_h, tt_h, ct_h, st_h)
    outs = (o0, o1, o2, o3, o4)
    bufs = (rows_a, rows_b)
    sems = (sem_a, sem_b)
    for t in range(5):
      pltpu.sync_copy(idxs[t].at[pl.ds(base, bpw)], idx_v.at[t])

    def fire(t):
      buf = bufs[t % 2]
      sem = sems[t % 2]
      handles = []
      for j in range(nch):
        handles.append(
            pltpu.async_copy(tabs[t].at[idx_v.at[t, pl.ds(j * _CH, _CH)]],
                             buf.at[pl.ds(j * _CH, _CH)], sem))
      return handles

    pending = fire(0)
    for t in range(5):
      for h in pending:
        h.wait()
      cur = bufs[t % 2]
      if t + 1 < 5:
        pending = fire(t + 1)
      pltpu.sync_copy(cur, outs[t].at[pl.ds(base, bpw)])

  return gather_kernel(item, time_i, rel, cat, store,
                       item_table, time_table, cat_table, store_table)


def _pop_body(ph_ref, tf_ref, out_ref):
  ph = ph_ref[...]
  tfr = tf_ref[...]
  tb = jnp.concatenate([tfr[r:r + 1, :].T for r in range(_OR)], axis=0)
  tb = jnp.maximum(tb - 1.0, 0.0)
  k = lax.broadcasted_iota(jnp.int32, (1, _T), 1).astype(jnp.float32)
  row = jnp.exp2(tb * _L2A)
  col = jnp.exp2(k * (-_L2A))
  w = row * col
  coef = jnp.where(k > tb, 0.0, jnp.where(k == 0.0, w, _ALPHA * w))
  pop = jnp.sum(coef * ph, axis=1, keepdims=True)
  out_ref[...] = pop.reshape(_OR, 128)


def _fold(v):
  return v[:, 0:_D] + v[:, _D:2 * _D]


def _stats_body(ie_ref, te_ref, re_ref, ce_ref, se_ref, out_ref, acc_ref):
  b = pl.program_id(0)

  @pl.when(b == 0)
  def _init():
    acc_ref[...] = jnp.zeros_like(acc_ref)

  ie = ie_ref[...]
  te = te_ref[...]
  re = re_ref[...]
  ce = ce_ref[...]
  se = se_ref[...]
  gap = re - te
  cols = (ie, te, re, ce, se)
  sums = [_fold(jnp.sum(x, axis=0, keepdims=True)) for x in cols]
  sqs = [_fold(jnp.sum(x * x, axis=0, keepdims=True)) for x in cols]
  gsum = _fold(jnp.sum(gap, axis=0, keepdims=True))
  gsq = _fold(jnp.sum(gap * gap, axis=0, keepdims=True))
  z = jnp.zeros((4, _D), jnp.float32)
  acc_ref[...] += jnp.concatenate(sums + sqs + [gsum, gsq, z], axis=0)

  @pl.when(b == _NB - 1)
  def _emit():
    out_ref[...] = acc_ref[...]


def _final_body(st_ref, pop_ref, ie_ref, te_ref, re_ref, ce_ref, se_ref,
                gt_ref, bt_ref, wt_ref, bt0_ref,
                gs_ref, bs_ref, ws_ref, bs0_ref, aw_ref,
                opop_ref, otime_ref, oside_ref, oout_ref):
  st = st_ref[...]
  inv_n = 1.0 / _B
  mu = st[0:5] * inv_n
  ex2 = st[5:10] * inv_n
  mu_g = st[10:11] * inv_n
  ex2_g = st[11:12] * inv_n
  var = ex2 - mu * mu
  var_g = ex2_g - mu_g * mu_g
  mu_t = jnp.concatenate([mu_g, mu[0:1], mu[1:2], mu[2:3]], axis=0)
  var_t = jnp.concatenate([var_g, var[0:1], var[1:2], var[2:3]], axis=0)
  w4 = wt_ref[...]
  v4 = gt_ref[...] * w4 * lax.rsqrt(var_t + _EPS)
  c_t = jnp.sum(bt_ref[...] * w4) - jnp.sum(mu_t * v4) + bt0_ref[...]
  w2 = ws_ref[...]
  v2 = gs_ref[...] * w2 * lax.rsqrt(var[3:5] + _EPS)
  c_s = jnp.sum(bs_ref[...] * w2) - jnp.sum(mu[3:5] * v2) + bs0_ref[...]
  aw = aw_ref[...]
  e = jnp.exp(aw - jnp.max(aw))
  wsm = e / jnp.sum(e)
  wa = wsm[0:1, 0:1]
  wb = wsm[0:1, 1:2]
  wc = wsm[0:1, 2:3]

  vv4 = jnp.concatenate([v4, v4], axis=1)
  vv2 = jnp.concatenate([v2, v2], axis=1)
  ie = ie_ref[...]
  te = te_ref[...]
  re = re_ref[...]
  gap = re - te
  yt = (gap * vv4[0:1] + ie * vv4[1:2] + te * vv4[2:3] + re * vv4[3:4])
  ys = ce_ref[...] * vv2[0:1] + se_ref[...] * vv2[1:2]
  dte = jnp.sum(yt[:, 0:_D], axis=1, keepdims=True)
  dto = jnp.sum(yt[:, _D:], axis=1, keepdims=True)
  dse = jnp.sum(ys[:, 0:_D], axis=1, keepdims=True)
  dso = jnp.sum(ys[:, _D:], axis=1, keepdims=True)

  lane = lax.broadcasted_iota(jnp.int32, (_D, 128), 1)
  sub = lax.broadcasted_iota(jnp.int32, (_D, 128), 0)
  pe = (lane == 2 * sub).astype(jnp.float32)
  po = (lane == 2 * sub + 1).astype(jnp.float32)
  dn = (((1,), (0,)), ((), ()))

  def interleave(ev, od):
    evq = ev.reshape(_OR, _D)
    odq = od.reshape(_OR, _D)
    return lax.dot_general(evq, pe, dn) + lax.dot_general(odq, po, dn)

  dt = interleave(dte, dto)
  ds = interleave(dse, dso)

  t_out = jnp.maximum(dt + c_t, 0.0)
  s_out = ds + c_s
  w_pop = pop_ref[...] * wa
  w_time = t_out * wb
  w_side = s_out * wc
  opop_ref[...] = w_pop
  otime_ref[...] = w_time
  oside_ref[...] = w_side
  oout_ref[...] = w_pop + w_time + w_side


def kernel(pop_history, item, time, release_time, category, store,
           item_table, time_table, cat_table, store_table,
           gamma_time, beta_time, W_time, b_time,
           gamma_side, beta_side, W_side, b_side, attn_w):
  i32 = jnp.int32
  item_e, time_e, rel_e, cat_e, store_e = _sc_gather(
      item.astype(i32), time.astype(i32), release_time.astype(i32),
      category.astype(i32), store.astype(i32),
      item_table, time_table, cat_table, store_table)
  packed = [x.reshape(_B // 2, 128)
            for x in (item_e, time_e, rel_e, cat_e, store_e)]

  tf = time.astype(jnp.float32).reshape(128, 128)
  pop_pk = pl.pallas_call(
      _pop_body,
      grid=(_NB,),
      in_specs=[
          pl.BlockSpec((_BLK, _T), lambda b: (b, 0)),
          pl.BlockSpec((_OR, 128), lambda b: (b, 0)),
      ],
      out_specs=pl.BlockSpec((_OR, 128), lambda b: (b, 0)),
      out_shape=jax.ShapeDtypeStruct((128, 128), jnp.float32),
  )(pop_history, tf)

  blk = lambda b: (b, 0)
  full = lambda b: (0, 0)
  espec = pl.BlockSpec((_PR, 128), blk)
  stats = pl.pallas_call(
      _stats_body,
      grid=(_NB,),
      in_specs=[espec] * 5,
      out_specs=pl.BlockSpec((16, _D), full),
      out_shape=jax.ShapeDtypeStruct((16, _D), jnp.float32),
      scratch_shapes=[pltpu.VMEM((16, _D), jnp.float32)],
  )(*packed)

  gt = gamma_time.reshape(4, _D)
  bt = beta_time.reshape(4, _D)
  wt = W_time.reshape(4, _D)
  bt0 = b_time.reshape(1, 1)
  gs = gamma_side.reshape(2, _D)
  bs = beta_side.reshape(2, _D)
  ws = W_side.reshape(2, _D)
  bs0 = b_side.reshape(1, 1)
  aw = attn_w.reshape(1, 3)

  out_spec = pl.BlockSpec((_OR, 128), blk)
  w_pop, w_time, w_side, output = pl.pallas_call(
      _final_body,
      grid=(_NB,),
      in_specs=[
          pl.BlockSpec((16, _D), full),
          pl.BlockSpec((_OR, 128), blk),
          espec, espec, espec, espec, espec,
          pl.BlockSpec((4, _D), full),
          pl.BlockSpec((4, _D), full),
          pl.BlockSpec((4, _D), full),
          pl.BlockSpec((1, 1), full),
          pl.BlockSpec((2, _D), full),
          pl.BlockSpec((2, _D), full),
          pl.BlockSpec((2, _D), full),
          pl.BlockSpec((1, 1), full),
          pl.BlockSpec((1, 3), full),
      ],
      out_specs=[out_spec] * 4,
      out_shape=[jax.ShapeDtypeStruct((128, 128), jnp.float32)] * 4,
  )(stats, pop_pk, *packed,
    gt, bt, wt, bt0, gs, bs, ws, bs0, aw)

  return tuple(o.reshape(_B, 1) for o in (w_pop, w_time, w_side, output))
